# Initial kernel scaffold; baseline (speedup 1.0000x reference)
import jax
import jax.numpy as jnp
from jax.experimental import pallas as pl


def kernel(x, edge_index, W1, b1, W2, b2, W3, b3):
    raise NotImplementedError("write your pallas kernel here")


# speedup vs baseline: 46.6448x; 46.6448x over previous
import functools

import jax
import jax.numpy as jnp
from jax import lax
from jax.experimental import pallas as pl
from jax.experimental.pallas import tpu as pltpu
from jax.experimental.pallas import tpu_sc as plsc

NCORES = 2
NSUB = 16
NW = NCORES * NSUB
CHUNK = 128
GRP = 4
NBUF = 2 * GRP


@functools.lru_cache(maxsize=None)
def _make_agg(n_pad: int, k: int, f: int):
    rows_per_w = n_pad // NSUB
    kc = k * CHUNK
    mesh = plsc.VectorSubcoreMesh(core_axis_name="c", subcore_axis_name="s")

    @functools.partial(
        pl.kernel,
        out_type=jax.ShapeDtypeStruct((NCORES, n_pad, f), jnp.float32),
        mesh=mesh,
        scratch_types=[
            pltpu.VMEM((kc,), jnp.int32),
            pltpu.VMEM((kc,), jnp.int32),
            pltpu.VMEM((NBUF, CHUNK, f), jnp.float32),
            pltpu.VMEM_SHARED((n_pad, f), jnp.float32),
            pltpu.VMEM_SHARED((n_pad, f), jnp.float32),
            pltpu.SemaphoreType.DMA((NBUF,)),
            pltpu.SemaphoreType.DMA((NBUF,)),
        ],
        compiler_params=pltpu.CompilerParams(use_tc_tiling_on_sc=False),
    )
    def agg(src_hbm, dst_hbm, g_hbm, zeros_hbm, out_hbm,
            src_v, dst_v, rows_v, acc, g_sh, gsem, ssem):
        c = lax.axis_index("c")
        s = lax.axis_index("s")
        wid = c * NSUB + s
        r0 = s * rows_per_w
        pltpu.sync_copy(zeros_hbm.at[pl.ds(r0, rows_per_w)],
                        acc.at[pl.ds(r0, rows_per_w)])
        pltpu.sync_copy(g_hbm.at[pl.ds(r0, rows_per_w)],
                        g_sh.at[pl.ds(r0, rows_per_w)])
        pltpu.sync_copy(src_hbm.at[pl.ds(wid * kc, kc)], src_v)
        pltpu.sync_copy(dst_hbm.at[pl.ds(wid * kc, kc)], dst_v)
        plsc.subcore_barrier()

        def idx(ref, j):
            return ref.at[pl.ds(j * CHUNK, CHUNK)]

        def slot_refs(half):
            return [(rows_v.at[half * GRP + i],
                     gsem.at[half * GRP + i], ssem.at[half * GRP + i])
                    for i in range(GRP)]

        def fire_gathers(half, jbase):
            for i, (rv, gs, _) in enumerate(slot_refs(half)):
                pltpu.async_copy(g_sh.at[idx(src_v, jbase + i)], rv, gs)

        def do_group(half, jbase, *, first, last):
            for i, (rv, gs, ss) in enumerate(slot_refs(half)):
                pltpu.make_async_copy(g_sh.at[idx(src_v, jbase + i)],
                                      rv, gs).wait()
                pltpu.async_copy(rv, acc.at[idx(dst_v, jbase + i)], ss,
                                 add=True)
            if not last:
                other = 1 - half
                for i, (rv, gs, ss) in enumerate(slot_refs(other)):
                    if not first:
                        pltpu.make_async_copy(
                            rv, acc.at[idx(dst_v, jbase + i)], ss).wait()
                    pltpu.async_copy(
                        g_sh.at[idx(src_v, jbase + GRP + i)], rv, gs)

        ngroups = k // GRP
        fire_gathers(0, 0)
        do_group(0, 0, first=True, last=False)

        def body(t, carry):
            jb = (2 * t + 1) * GRP
            do_group(1, jb, first=False, last=False)
            do_group(0, jb + GRP, first=False, last=False)
            return carry

        lax.fori_loop(0, (ngroups - 2) // 2, body, 0)
        do_group(1, (ngroups - 1) * GRP, first=False, last=True)
        for half in (0, 1):
            for i, (rv, _, ss) in enumerate(slot_refs(half)):
                jb = (ngroups - 2 + half) * GRP
                pltpu.make_async_copy(rv, acc.at[idx(dst_v, jb + i)],
                                      ss).wait()
        plsc.subcore_barrier()
        pltpu.sync_copy(acc.at[pl.ds(r0, rows_per_w)],
                        out_hbm.at[c, pl.ds(r0, rows_per_w)])

    return agg


DEG_W = 16


@functools.lru_cache(maxsize=None)
def _make_deg(n_pad: int, k: int):
    rows_per_w = n_pad // NSUB
    kc = k * CHUNK
    mesh = plsc.VectorSubcoreMesh(core_axis_name="c", subcore_axis_name="s")

    @functools.partial(
        pl.kernel,
        out_type=jax.ShapeDtypeStruct((NCORES, n_pad, DEG_W), jnp.float32),
        mesh=mesh,
        scratch_types=[
            pltpu.VMEM((kc,), jnp.int32),
            pltpu.VMEM((CHUNK, DEG_W), jnp.float32),
            pltpu.VMEM_SHARED((n_pad, DEG_W), jnp.float32),
        ],
        compiler_params=pltpu.CompilerParams(use_tc_tiling_on_sc=False),
    )
    def deg(dst_hbm, ones_hbm, zeros_hbm, out_hbm, dst_v, ones_v, acc):
        c = lax.axis_index("c")
        s = lax.axis_index("s")
        wid = c * NSUB + s
        r0 = s * rows_per_w
        pltpu.sync_copy(zeros_hbm.at[pl.ds(r0, rows_per_w)],
                        acc.at[pl.ds(r0, rows_per_w)])
        pltpu.sync_copy(dst_hbm.at[pl.ds(wid * kc, kc)], dst_v)
        pltpu.sync_copy(ones_hbm, ones_v)
        plsc.subcore_barrier()

        def body(j, carry):
            pltpu.sync_copy(ones_v, acc.at[dst_v.at[pl.ds(j * CHUNK, CHUNK)]],
                            add=True)
            return carry

        lax.fori_loop(0, k, body, 0)
        plsc.subcore_barrier()
        pltpu.sync_copy(acc.at[pl.ds(r0, rows_per_w)],
                        out_hbm.at[c, pl.ds(r0, rows_per_w)])

    return deg


def _tc_mm_body(x_ref, w1_ref, h_ref):
    h_ref[...] = jnp.dot(x_ref[...], w1_ref[...],
                         preferred_element_type=jnp.float32)


def _tc1_body(degp_ref, h_ref, g1_ref, dinv_ref, *, n: int):
    deg = degp_ref[0] + degp_ref[1] + 1.0
    dinv = lax.rsqrt(deg)
    g1_ref[:n, :] = h_ref[...] * dinv[:n]
    g1_ref[n:, :] = jnp.zeros_like(g1_ref[n:, :])
    dinv_ref[...] = dinv


def _tc_mid_body(p_ref, g_ref, dinv_ref, w_ref, b_ref, gout_ref):
    agg = p_ref[0] + p_ref[1] + g_ref[...]
    xn = jnp.maximum(agg * dinv_ref[...] + b_ref[...], 0.0)
    gout_ref[...] = jnp.dot(xn, w_ref[...],
                            preferred_element_type=jnp.float32) * dinv_ref[...]


def _tc4_body(p_ref, g_ref, dinv_ref, b3_ref, out_ref, *, n: int, c: int):
    agg = p_ref[0, :n] + p_ref[1, :n] + g_ref[:n]
    z = agg[:, :c] * dinv_ref[:n] + b3_ref[...]
    m = jnp.max(z, axis=1, keepdims=True)
    zs = z - m
    lse = jnp.log(jnp.sum(jnp.exp(zs), axis=1, keepdims=True))
    out_ref[...] = zs - lse


def kernel(x, edge_index, W1, b1, W2, b2, W3, b3):
    n, d = x.shape
    e = edge_index.shape[1]
    h1 = W1.shape[1]
    h2 = W2.shape[1]
    c = W3.shape[1]
    fpad = max(h2, 16)

    n_pad = -(-n // 256) * 256
    k = -(-e // (NW * CHUNK))
    k = -(-k // 8) * 8
    e_pad = k * NW * CHUNK
    pad_idx = n_pad - 8

    ei = edge_index.astype(jnp.int32)
    pad = jnp.full((e_pad - e,), pad_idx, jnp.int32)
    src = jnp.concatenate([ei[0], pad])
    dst = jnp.concatenate([ei[1], pad])

    zeros_deg = jnp.zeros((n_pad, DEG_W), jnp.float32)
    ones = jnp.ones((CHUNK, DEG_W), jnp.float32)

    h = pl.pallas_call(
        _tc_mm_body,
        out_shape=jax.ShapeDtypeStruct((n, h1), jnp.float32),
    )(x, W1)

    degp = _make_deg(n_pad, k)(dst, ones, zeros_deg)[:, :, :1]

    g1, dinv = pl.pallas_call(
        functools.partial(_tc1_body, n=n),
        out_shape=[jax.ShapeDtypeStruct((n_pad, h1), jnp.float32),
                   jax.ShapeDtypeStruct((n_pad, 1), jnp.float32)],
    )(degp, h)

    p1 = _make_agg(n_pad, k, h1)(src, dst, g1, jnp.zeros((n_pad, h1), jnp.float32))
    g2 = pl.pallas_call(
        _tc_mid_body,
        out_shape=jax.ShapeDtypeStruct((n_pad, h2), jnp.float32),
    )(p1, g1, dinv, W2, b1.reshape(1, h1))

    zeros_f = jnp.zeros((n_pad, fpad), jnp.float32)
    p2 = _make_agg(n_pad, k, h2)(src, dst, g2, zeros_f[:, :h2])
    w3p = jnp.zeros((h2, fpad), jnp.float32).at[:, :c].set(W3)
    g3 = pl.pallas_call(
        _tc_mid_body,
        out_shape=jax.ShapeDtypeStruct((n_pad, fpad), jnp.float32),
    )(p2, g2, dinv, w3p, b2.reshape(1, h2))

    p3 = _make_agg(n_pad, k, fpad)(src, dst, g3, zeros_f)
    out = pl.pallas_call(
        functools.partial(_tc4_body, n=n, c=c),
        out_shape=jax.ShapeDtypeStruct((n, c), jnp.float32),
    )(p3, g3, dinv, b3.reshape(1, c))

    return out


# speedup vs baseline: 63.5032x; 1.3614x over previous
import functools

import jax
import jax.numpy as jnp
from jax import lax
from jax.experimental import pallas as pl
from jax.experimental.pallas import tpu as pltpu
from jax.experimental.pallas import tpu_sc as plsc

NCORES = 2
NSUB = 16
NW = NCORES * NSUB
CHUNK = 128
GRP = 4
NBUF = 2 * GRP


@functools.lru_cache(maxsize=None)
def _make_agg(n_pad: int, k: int, f: int):
    rows_per_w = n_pad // NSUB
    kc = k * CHUNK
    mesh = plsc.VectorSubcoreMesh(core_axis_name="c", subcore_axis_name="s")

    @functools.partial(
        pl.kernel,
        out_type=jax.ShapeDtypeStruct((NCORES, n_pad, f), jnp.float32),
        mesh=mesh,
        scratch_types=[
            pltpu.VMEM((kc,), jnp.int32),
            pltpu.VMEM((kc,), jnp.int32),
            pltpu.VMEM((NBUF, CHUNK, f), jnp.float32),
            pltpu.VMEM_SHARED((n_pad, f), jnp.float32),
            pltpu.VMEM_SHARED((n_pad, f), jnp.float32),
            pltpu.SemaphoreType.DMA((NBUF,)),
            pltpu.SemaphoreType.DMA((NBUF,)),
        ],
        compiler_params=pltpu.CompilerParams(use_tc_tiling_on_sc=False),
    )
    def agg(src_hbm, dst_hbm, g_hbm, zeros_hbm, out_hbm,
            src_v, dst_v, rows_v, acc, g_sh, gsem, ssem):
        c = lax.axis_index("c")
        s = lax.axis_index("s")
        wid = c * NSUB + s
        r0 = s * rows_per_w
        pltpu.sync_copy(zeros_hbm.at[pl.ds(r0, rows_per_w)],
                        acc.at[pl.ds(r0, rows_per_w)])
        pltpu.sync_copy(g_hbm.at[pl.ds(r0, rows_per_w)],
                        g_sh.at[pl.ds(r0, rows_per_w)])
        pltpu.sync_copy(src_hbm.at[pl.ds(wid * kc, kc)], src_v)
        pltpu.sync_copy(dst_hbm.at[pl.ds(wid * kc, kc)], dst_v)
        plsc.subcore_barrier()

        def idx(ref, j):
            return ref.at[pl.ds(j * CHUNK, CHUNK)]

        def slot_refs(half):
            return [(rows_v.at[half * GRP + i],
                     gsem.at[half * GRP + i], ssem.at[half * GRP + i])
                    for i in range(GRP)]

        def fire_gathers(half, jbase):
            for i, (rv, gs, _) in enumerate(slot_refs(half)):
                pltpu.async_copy(g_sh.at[idx(src_v, jbase + i)], rv, gs)

        def do_group(half, jbase, *, first, last):
            for i, (rv, gs, ss) in enumerate(slot_refs(half)):
                pltpu.make_async_copy(g_sh.at[idx(src_v, jbase + i)],
                                      rv, gs).wait()
                pltpu.async_copy(rv, acc.at[idx(dst_v, jbase + i)], ss,
                                 add=True)
            if not last:
                other = 1 - half
                for i, (rv, gs, ss) in enumerate(slot_refs(other)):
                    if not first:
                        pltpu.make_async_copy(
                            rv, acc.at[idx(dst_v, jbase + i)], ss).wait()
                    pltpu.async_copy(
                        g_sh.at[idx(src_v, jbase + GRP + i)], rv, gs)

        ngroups = k // GRP
        fire_gathers(0, 0)
        do_group(0, 0, first=True, last=False)

        def body(t, carry):
            jb = (2 * t + 1) * GRP
            do_group(1, jb, first=False, last=False)
            do_group(0, jb + GRP, first=False, last=False)
            return carry

        lax.fori_loop(0, (ngroups - 2) // 2, body, 0)
        do_group(1, (ngroups - 1) * GRP, first=False, last=True)
        for half in (0, 1):
            for i, (rv, _, ss) in enumerate(slot_refs(half)):
                jb = (ngroups - 2 + half) * GRP
                pltpu.make_async_copy(rv, acc.at[idx(dst_v, jb + i)],
                                      ss).wait()
        plsc.subcore_barrier()
        pltpu.sync_copy(acc.at[pl.ds(r0, rows_per_w)],
                        out_hbm.at[c, pl.ds(r0, rows_per_w)])

    return agg


DEG_W = 16


@functools.lru_cache(maxsize=None)
def _make_deg(n_pad: int, k: int):
    rows_per_w = n_pad // NSUB
    kc = k * CHUNK
    mesh = plsc.VectorSubcoreMesh(core_axis_name="c", subcore_axis_name="s")

    @functools.partial(
        pl.kernel,
        out_type=jax.ShapeDtypeStruct((NCORES, n_pad, DEG_W), jnp.float32),
        mesh=mesh,
        scratch_types=[
            pltpu.VMEM((kc,), jnp.int32),
            pltpu.VMEM((CHUNK, DEG_W), jnp.float32),
            pltpu.VMEM_SHARED((n_pad, DEG_W), jnp.float32),
        ],
        compiler_params=pltpu.CompilerParams(use_tc_tiling_on_sc=False),
    )
    def deg(dst_hbm, ones_hbm, zeros_hbm, out_hbm, dst_v, ones_v, acc):
        c = lax.axis_index("c")
        s = lax.axis_index("s")
        wid = c * NSUB + s
        r0 = s * rows_per_w
        pltpu.sync_copy(zeros_hbm.at[pl.ds(r0, rows_per_w)],
                        acc.at[pl.ds(r0, rows_per_w)])
        pltpu.sync_copy(dst_hbm.at[pl.ds(wid * kc, kc)], dst_v)
        pltpu.sync_copy(ones_hbm, ones_v)
        plsc.subcore_barrier()

        def body(j, carry):
            pltpu.sync_copy(ones_v, acc.at[dst_v.at[pl.ds(j * CHUNK, CHUNK)]],
                            add=True)
            return carry

        lax.fori_loop(0, k, body, 0)
        plsc.subcore_barrier()
        pltpu.sync_copy(acc.at[pl.ds(r0, rows_per_w)],
                        out_hbm.at[c, pl.ds(r0, rows_per_w)])

    return deg


def _tc_mm_body(x_ref, w1_ref, h_ref):
    h_ref[...] = jnp.dot(x_ref[...], w1_ref[...],
                         preferred_element_type=jnp.float32)


def _pack4(v):
    r = v.shape[0] // 4
    v4 = jnp.reshape(v, (r, 4, 32))
    return jnp.concatenate([v4[:, a, :] for a in range(4)], axis=1)


def _rows2(v):
    r = v.shape[0] // 2
    v2 = jnp.reshape(v, (r, 2, 64))
    return jnp.concatenate([v2[:, 0, :], v2[:, 1, :]], axis=1)


def _tc1_body(degp_ref, h_ref, g1p_ref, dv32_ref, dv16_ref, *, n: int):
    deg = degp_ref[0] + degp_ref[1]
    dv16 = lax.rsqrt(deg + 1.0)
    dv16_ref[...] = dv16
    ev = jnp.concatenate(
        [dv16[:, 16 * j:16 * j + 16] for j in (0, 0, 1, 1, 2, 2, 3, 3)],
        axis=1)
    od = jnp.concatenate(
        [dv16[:, 16 * j:16 * j + 16] for j in (4, 4, 5, 5, 6, 6, 7, 7)],
        axis=1)
    dv32 = jnp.reshape(jnp.stack([ev, od], axis=1), dv32_ref.shape)
    dv32_ref[...] = dv32
    rows = n * 32 // 128
    g1p_ref[:rows, :] = _pack4(h_ref[...]) * dv32[:rows]
    g1p_ref[rows:, :] = jnp.zeros_like(g1p_ref[rows:, :])


def _tc_mid_body(p_ref, g_ref, dvin_ref, dvout_ref, w_ref, b_ref, gout_ref):
    agg = p_ref[0] + p_ref[1] + g_ref[...]
    xn = jnp.maximum(agg * dvin_ref[...] + b_ref[...], 0.0)
    y = jnp.dot(xn, w_ref[...], preferred_element_type=jnp.float32)
    if y.shape != gout_ref.shape:
        y = _rows2(y)
    gout_ref[...] = y * dvout_ref[...]


def _tc4_body(p_ref, g_ref, dv16_ref, b3c_ref, out_ref, *, c: int):
    s = (p_ref[0] + p_ref[1] + g_ref[...]) * dv16_ref[...]
    zc = jnp.concatenate([s[:, 16 * j:16 * j + c] for j in range(8)], axis=1)
    zc = zc + b3c_ref[...]
    w = zc.shape[1]
    gi = lax.broadcasted_iota(jnp.int32, (w, w), 0) // c
    gj = lax.broadcasted_iota(jnp.int32, (w, w), 1) // c
    gm = (gi == gj).astype(jnp.float32)
    zs = zc - jnp.dot(zc, gm * (1.0 / c), preferred_element_type=jnp.float32)
    lse = jnp.log(jnp.dot(jnp.exp(zs), gm, preferred_element_type=jnp.float32))
    nrows = out_ref.shape[0]
    out_ref[...] = (zs - lse)[:nrows]


def kernel(x, edge_index, W1, b1, W2, b2, W3, b3):
    n, d = x.shape
    e = edge_index.shape[1]
    h1 = W1.shape[1]
    h2 = W2.shape[1]
    c = W3.shape[1]
    fpad = max(h2, 16)

    n_pad = -(-n // 256) * 256
    k = -(-e // (NW * CHUNK))
    k = -(-k // 8) * 8
    e_pad = k * NW * CHUNK
    pad_idx = n_pad - 8

    ei = edge_index.astype(jnp.int32)
    pad = jnp.full((e_pad - e,), pad_idx, jnp.int32)
    src = jnp.concatenate([ei[0], pad])
    dst = jnp.concatenate([ei[1], pad])

    zeros_deg = jnp.zeros((n_pad, DEG_W), jnp.float32)
    ones = jnp.ones((CHUNK, DEG_W), jnp.float32)

    h = pl.pallas_call(
        _tc_mm_body,
        out_shape=jax.ShapeDtypeStruct((n, h1), jnp.float32),
    )(x, W1)

    r32 = n_pad * h1 // 128
    r16 = n_pad * fpad // 128
    rdeg = n_pad * DEG_W // 128
    w2bd = jnp.kron(jnp.eye(128 // h1, dtype=jnp.float32), W2)
    w3p = jnp.zeros((h2, fpad), jnp.float32).at[:, :c].set(W3)
    w3bd = jnp.kron(jnp.eye(128 // h2, dtype=jnp.float32), w3p)
    b1p = jnp.tile(b1, 128 // h1).reshape(1, 128)
    b2p = jnp.tile(b2, 128 // h2).reshape(1, 128)

    degp = _make_deg(n_pad, k)(dst, ones, zeros_deg)

    g1p, dv32, dv16 = pl.pallas_call(
        functools.partial(_tc1_body, n=n),
        out_shape=[jax.ShapeDtypeStruct((r32, 128), jnp.float32),
                   jax.ShapeDtypeStruct((r32, 128), jnp.float32),
                   jax.ShapeDtypeStruct((r16, 128), jnp.float32)],
    )(degp.reshape(NCORES, rdeg, 128), h)

    p1 = _make_agg(n_pad, k, h1)(src, dst, g1p.reshape(n_pad, h1),
                                 jnp.zeros((n_pad, h1), jnp.float32))
    g2p = pl.pallas_call(
        _tc_mid_body,
        out_shape=jax.ShapeDtypeStruct((r16, 128), jnp.float32),
    )(p1.reshape(NCORES, r32, 128), g1p, dv32, dv16, w2bd, b1p)

    zeros_f = jnp.zeros((n_pad, fpad), jnp.float32)
    p2 = _make_agg(n_pad, k, h2)(src, dst, g2p.reshape(n_pad, h2),
                                 zeros_f[:, :h2])
    g3p = pl.pallas_call(
        _tc_mid_body,
        out_shape=jax.ShapeDtypeStruct((r16, 128), jnp.float32),
    )(p2.reshape(NCORES, r16, 128), g2p, dv16, dv16, w3bd, b2p)

    p3 = _make_agg(n_pad, k, fpad)(src, dst, g3p.reshape(n_pad, fpad), zeros_f)
    b3c = jnp.tile(b3, 32 // c).reshape(1, 32)
    out = pl.pallas_call(
        functools.partial(_tc4_body, c=c),
        out_shape=jax.ShapeDtypeStruct((n * c // 32, 32), jnp.float32),
    )(p3.reshape(NCORES, r16, 128), g3p, dv16, b3c)

    return out.reshape(n, c)


# speedup vs baseline: 65.6228x; 1.0334x over previous
import functools

import jax
import jax.numpy as jnp
from jax import lax
from jax.experimental import pallas as pl
from jax.experimental.pallas import tpu as pltpu
from jax.experimental.pallas import tpu_sc as plsc

NCORES = 2
NSUB = 16
NW = NCORES * NSUB
CHUNK = 128
GRP = 4
NBUF = 2 * GRP


@functools.lru_cache(maxsize=None)
def _make_agg(n_pad: int, k: int, f: int):
    rows_per_w = n_pad // NSUB
    kc = k * CHUNK
    mesh = plsc.VectorSubcoreMesh(core_axis_name="c", subcore_axis_name="s")

    @functools.partial(
        pl.kernel,
        out_type=jax.ShapeDtypeStruct((NCORES, n_pad, f), jnp.float32),
        mesh=mesh,
        scratch_types=[
            pltpu.VMEM((kc,), jnp.int32),
            pltpu.VMEM((kc,), jnp.int32),
            pltpu.VMEM((NBUF, CHUNK, f), jnp.float32),
            pltpu.VMEM_SHARED((n_pad, f), jnp.float32),
            pltpu.VMEM_SHARED((n_pad, f), jnp.float32),
            pltpu.SemaphoreType.DMA((NBUF,)),
            pltpu.SemaphoreType.DMA((NBUF,)),
        ],
        compiler_params=pltpu.CompilerParams(use_tc_tiling_on_sc=False),
    )
    def agg(src_hbm, dst_hbm, g_hbm, zeros_hbm, out_hbm,
            src_v, dst_v, rows_v, acc, g_sh, gsem, ssem):
        c = lax.axis_index("c")
        s = lax.axis_index("s")
        wid = c * NSUB + s
        r0 = s * rows_per_w
        stage = [
            (zeros_hbm.at[pl.ds(r0, rows_per_w)], acc.at[pl.ds(r0, rows_per_w)]),
            (g_hbm.at[pl.ds(r0, rows_per_w)], g_sh.at[pl.ds(r0, rows_per_w)]),
            (src_hbm.at[pl.ds(wid * kc, kc)], src_v),
            (dst_hbm.at[pl.ds(wid * kc, kc)], dst_v),
        ]
        for i, (a, b) in enumerate(stage):
            pltpu.async_copy(a, b, gsem.at[i])
        for i, (a, b) in enumerate(stage):
            pltpu.make_async_copy(a, b, gsem.at[i]).wait()
        plsc.subcore_barrier()

        def idx(ref, j):
            return ref.at[pl.ds(j * CHUNK, CHUNK)]

        def slot_refs(half):
            return [(rows_v.at[half * GRP + i],
                     gsem.at[half * GRP + i], ssem.at[half * GRP + i])
                    for i in range(GRP)]

        def fire_gathers(half, jbase):
            for i, (rv, gs, _) in enumerate(slot_refs(half)):
                pltpu.async_copy(g_sh.at[idx(src_v, jbase + i)], rv, gs)

        def do_group(half, jbase, *, first, last):
            for i, (rv, gs, ss) in enumerate(slot_refs(half)):
                pltpu.make_async_copy(g_sh.at[idx(src_v, jbase + i)],
                                      rv, gs).wait()
                pltpu.async_copy(rv, acc.at[idx(dst_v, jbase + i)], ss,
                                 add=True)
            if not last:
                other = 1 - half
                for i, (rv, gs, ss) in enumerate(slot_refs(other)):
                    if not first:
                        pltpu.make_async_copy(
                            rv, acc.at[idx(dst_v, jbase + i)], ss).wait()
                    pltpu.async_copy(
                        g_sh.at[idx(src_v, jbase + GRP + i)], rv, gs)

        ngroups = k // GRP
        fire_gathers(0, 0)
        do_group(0, 0, first=True, last=False)

        def body(t, carry):
            jb = (2 * t + 1) * GRP
            do_group(1, jb, first=False, last=False)
            do_group(0, jb + GRP, first=False, last=False)
            return carry

        lax.fori_loop(0, (ngroups - 2) // 2, body, 0)
        do_group(1, (ngroups - 1) * GRP, first=False, last=True)
        for half in (0, 1):
            for i, (rv, _, ss) in enumerate(slot_refs(half)):
                jb = (ngroups - 2 + half) * GRP
                pltpu.make_async_copy(rv, acc.at[idx(dst_v, jb + i)],
                                      ss).wait()
        plsc.subcore_barrier()
        pltpu.sync_copy(acc.at[pl.ds(r0, rows_per_w)],
                        out_hbm.at[c, pl.ds(r0, rows_per_w)])

    return agg


DEG_W = 16


@functools.lru_cache(maxsize=None)
def _make_deg(n_pad: int, k: int):
    rows_per_w = n_pad // NSUB
    kc = k * CHUNK
    mesh = plsc.VectorSubcoreMesh(core_axis_name="c", subcore_axis_name="s")

    @functools.partial(
        pl.kernel,
        out_type=jax.ShapeDtypeStruct((NCORES, n_pad, DEG_W), jnp.float32),
        mesh=mesh,
        scratch_types=[
            pltpu.VMEM((kc,), jnp.int32),
            pltpu.VMEM((CHUNK, DEG_W), jnp.float32),
            pltpu.VMEM_SHARED((n_pad, DEG_W), jnp.float32),
            pltpu.SemaphoreType.DMA((NBUF,)),
        ],
        compiler_params=pltpu.CompilerParams(use_tc_tiling_on_sc=False),
    )
    def deg(dst_hbm, ones_hbm, zeros_hbm, out_hbm, dst_v, ones_v, acc, dsem):
        c = lax.axis_index("c")
        s = lax.axis_index("s")
        wid = c * NSUB + s
        r0 = s * rows_per_w
        stage = [
            (zeros_hbm.at[pl.ds(r0, rows_per_w)], acc.at[pl.ds(r0, rows_per_w)]),
            (dst_hbm.at[pl.ds(wid * kc, kc)], dst_v),
            (ones_hbm, ones_v),
        ]
        for i, (a, b) in enumerate(stage):
            pltpu.async_copy(a, b, dsem.at[i])
        for i, (a, b) in enumerate(stage):
            pltpu.make_async_copy(a, b, dsem.at[i]).wait()
        plsc.subcore_barrier()

        def sc_at(j):
            return acc.at[dst_v.at[pl.ds(j * CHUNK, CHUNK)]]

        for i in range(NBUF):
            pltpu.async_copy(ones_v, sc_at(i), dsem.at[i], add=True)

        def body(j, carry):
            slot = lax.rem(j, NBUF)
            pltpu.make_async_copy(ones_v, sc_at(j - NBUF), dsem.at[slot]).wait()
            pltpu.async_copy(ones_v, sc_at(j), dsem.at[slot], add=True)
            return carry

        lax.fori_loop(NBUF, k, body, 0)

        def drain(j, carry):
            pltpu.make_async_copy(ones_v, sc_at(j), dsem.at[lax.rem(j, NBUF)]).wait()
            return carry

        lax.fori_loop(k - NBUF, k, drain, 0)
        plsc.subcore_barrier()
        pltpu.sync_copy(acc.at[pl.ds(r0, rows_per_w)],
                        out_hbm.at[c, pl.ds(r0, rows_per_w)])

    return deg


def _tc_mm_body(x_ref, w1_ref, h_ref):
    h_ref[...] = jnp.dot(x_ref[...], w1_ref[...],
                         preferred_element_type=jnp.float32)


def _pack4(v):
    r = v.shape[0] // 4
    v4 = jnp.reshape(v, (r, 4, 32))
    return jnp.concatenate([v4[:, a, :] for a in range(4)], axis=1)


def _rows2(v):
    r = v.shape[0] // 2
    v2 = jnp.reshape(v, (r, 2, 64))
    return jnp.concatenate([v2[:, 0, :], v2[:, 1, :]], axis=1)


def _tc1_body(degp_ref, h_ref, g1p_ref, dv32_ref, dv16_ref, *, n: int):
    deg = degp_ref[0] + degp_ref[1]
    dv16 = lax.rsqrt(deg + 1.0)
    dv16_ref[...] = dv16
    ev = jnp.concatenate(
        [dv16[:, 16 * j:16 * j + 16] for j in (0, 0, 1, 1, 2, 2, 3, 3)],
        axis=1)
    od = jnp.concatenate(
        [dv16[:, 16 * j:16 * j + 16] for j in (4, 4, 5, 5, 6, 6, 7, 7)],
        axis=1)
    dv32 = jnp.reshape(jnp.stack([ev, od], axis=1), dv32_ref.shape)
    dv32_ref[...] = dv32
    rows = n * 32 // 128
    g1p_ref[:rows, :] = _pack4(h_ref[...]) * dv32[:rows]
    g1p_ref[rows:, :] = jnp.zeros_like(g1p_ref[rows:, :])


def _tc_mid_body(p_ref, g_ref, dvin_ref, dvout_ref, w_ref, b_ref, gout_ref):
    agg = p_ref[0] + p_ref[1] + g_ref[...]
    xn = jnp.maximum(agg * dvin_ref[...] + b_ref[...], 0.0)
    y = jnp.dot(xn, w_ref[...], preferred_element_type=jnp.float32)
    if y.shape != gout_ref.shape:
        y = _rows2(y)
    gout_ref[...] = y * dvout_ref[...]


def _tc4_body(p_ref, g_ref, dv16_ref, b3c_ref, out_ref, *, c: int):
    s = (p_ref[0] + p_ref[1] + g_ref[...]) * dv16_ref[...]
    zc = jnp.concatenate([s[:, 16 * j:16 * j + c] for j in range(8)], axis=1)
    zc = zc + b3c_ref[...]
    w = zc.shape[1]
    gi = lax.broadcasted_iota(jnp.int32, (w, w), 0) // c
    gj = lax.broadcasted_iota(jnp.int32, (w, w), 1) // c
    gm = (gi == gj).astype(jnp.float32)
    zs = zc - jnp.dot(zc, gm * (1.0 / c), preferred_element_type=jnp.float32)
    lse = jnp.log(jnp.dot(jnp.exp(zs), gm, preferred_element_type=jnp.float32))
    nrows = out_ref.shape[0]
    out_ref[...] = (zs - lse)[:nrows]


def kernel(x, edge_index, W1, b1, W2, b2, W3, b3):
    n, d = x.shape
    e = edge_index.shape[1]
    h1 = W1.shape[1]
    h2 = W2.shape[1]
    c = W3.shape[1]
    fpad = max(h2, 16)

    n_pad = -(-n // 256) * 256
    k = -(-e // (NW * CHUNK))
    k = -(-k // 8) * 8
    e_pad = k * NW * CHUNK
    pad_idx = n_pad - 8

    ei = edge_index.astype(jnp.int32)
    pad = jnp.full((e_pad - e,), pad_idx, jnp.int32)
    src = jnp.concatenate([ei[0], pad])
    dst = jnp.concatenate([ei[1], pad])

    zeros_deg = jnp.zeros((n_pad, DEG_W), jnp.float32)
    ones = jnp.ones((CHUNK, DEG_W), jnp.float32)

    h = pl.pallas_call(
        _tc_mm_body,
        out_shape=jax.ShapeDtypeStruct((n, h1), jnp.float32),
    )(x, W1)

    r32 = n_pad * h1 // 128
    r16 = n_pad * fpad // 128
    rdeg = n_pad * DEG_W // 128
    w2bd = jnp.kron(jnp.eye(128 // h1, dtype=jnp.float32), W2)
    w3p = jnp.zeros((h2, fpad), jnp.float32).at[:, :c].set(W3)
    w3bd = jnp.kron(jnp.eye(128 // h2, dtype=jnp.float32), w3p)
    b1p = jnp.tile(b1, 128 // h1).reshape(1, 128)
    b2p = jnp.tile(b2, 128 // h2).reshape(1, 128)

    degp = _make_deg(n_pad, k)(dst, ones, zeros_deg)

    g1p, dv32, dv16 = pl.pallas_call(
        functools.partial(_tc1_body, n=n),
        out_shape=[jax.ShapeDtypeStruct((r32, 128), jnp.float32),
                   jax.ShapeDtypeStruct((r32, 128), jnp.float32),
                   jax.ShapeDtypeStruct((r16, 128), jnp.float32)],
    )(degp.reshape(NCORES, rdeg, 128), h)

    p1 = _make_agg(n_pad, k, h1)(src, dst, g1p.reshape(n_pad, h1),
                                 jnp.zeros((n_pad, h1), jnp.float32))
    g2p = pl.pallas_call(
        _tc_mid_body,
        out_shape=jax.ShapeDtypeStruct((r16, 128), jnp.float32),
    )(p1.reshape(NCORES, r32, 128), g1p, dv32, dv16, w2bd, b1p)

    zeros_f = jnp.zeros((n_pad, fpad), jnp.float32)
    p2 = _make_agg(n_pad, k, h2)(src, dst, g2p.reshape(n_pad, h2),
                                 zeros_f[:, :h2])
    g3p = pl.pallas_call(
        _tc_mid_body,
        out_shape=jax.ShapeDtypeStruct((r16, 128), jnp.float32),
    )(p2.reshape(NCORES, r16, 128), g2p, dv16, dv16, w3bd, b2p)

    p3 = _make_agg(n_pad, k, fpad)(src, dst, g3p.reshape(n_pad, fpad), zeros_f)
    b3c = jnp.tile(b3, 32 // c).reshape(1, 32)
    out = pl.pallas_call(
        functools.partial(_tc4_body, c=c),
        out_shape=jax.ShapeDtypeStruct((n * c // 32, 32), jnp.float32),
    )(p3.reshape(NCORES, r16, 128), g3p, dv16, b3c)

    return out.reshape(n, c)
